# baseline (speedup 1.0000x reference)
import jax
import jax.numpy as jnp
from jax import lax
from jax.experimental import pallas as pl
from jax.experimental.pallas import tpu as pltpu

F32 = jnp.float32
_BE = 2000
_BN = 2000


def _ln(x, s, b):
    mu = jnp.mean(x, axis=-1, keepdims=True)
    xc = x - mu
    var = jnp.mean(xc * xc, axis=-1, keepdims=True)
    return xc * lax.rsqrt(var + 1e-5) * s + b


def _mm(x, w):
    return jnp.dot(x, w, preferred_element_type=F32,
                   precision=lax.Precision.HIGHEST)


def _full(shape):
    return pl.BlockSpec(shape, lambda i: (0,) * len(shape))


def _rows(bs, w):
    return pl.BlockSpec((bs, w), lambda i: (i, 0))


def _node_enc(vel, ntype, w1v, wtype, w2, b2, w3, b3, lns, lnb):
    n = vel.shape[0]

    def body(vel_ref, t_ref, w1v_ref, wt_ref, w2_ref, b2_ref, w3_ref, b3_ref,
             s_ref, b_ref, o_ref):
        v = vel_ref[...]
        t = v[:, 0:1] * w1v_ref[0:1, :] + v[:, 1:2] * w1v_ref[1:2, :]
        tt = t_ref[...]
        for k in range(9):
            t = t + jnp.where(tt == k, 1.0, 0.0) * wt_ref[k:k + 1, :]
        t = jnp.maximum(t, 0.0)
        t = jnp.maximum(_mm(t, w2_ref[...]) + b2_ref[...], 0.0)
        t = _mm(t, w3_ref[...]) + b3_ref[...]
        o_ref[...] = _ln(t, s_ref[...], b_ref[...])

    return pl.pallas_call(
        body,
        grid=(n // _BN,),
        in_specs=[_rows(_BN, 2), _rows(_BN, 1), _full((2, 128)), _full((9, 128)),
                  _full((128, 128)), _full((1, 128)), _full((128, 128)),
                  _full((1, 128)), _full((1, 128)), _full((1, 128))],
        out_specs=_rows(_BN, 128),
        out_shape=jax.ShapeDtypeStruct((n, 128), F32),
    )(vel, ntype, w1v, wtype, w2, b2, w3, b3, lns, lnb)


def _edge_enc(sp, dp, w1, b1, w2, b2, w3, b3, lns, lnb):
    e = sp.shape[0]

    def body(sp_ref, dp_ref, w1_ref, b1_ref, w2_ref, b2_ref, w3_ref, b3_ref,
             s_ref, bb_ref, o_ref):
        r = sp_ref[...] - dp_ref[...]
        rx = r[:, 0:1]
        ry = r[:, 1:2]
        rn = jnp.sqrt(rx * rx + ry * ry)
        t = rx * w1_ref[0:1, :] + ry * w1_ref[1:2, :] + rn * w1_ref[2:3, :] + b1_ref[...]
        t = jnp.maximum(t, 0.0)
        t = jnp.maximum(_mm(t, w2_ref[...]) + b2_ref[...], 0.0)
        t = _mm(t, w3_ref[...]) + b3_ref[...]
        o_ref[...] = _ln(t, s_ref[...], bb_ref[...])

    return pl.pallas_call(
        body,
        grid=(e // _BE,),
        in_specs=[_rows(_BE, 2), _rows(_BE, 2), _full((3, 128)), _full((1, 128)),
                  _full((128, 128)), _full((1, 128)), _full((128, 128)),
                  _full((1, 128)), _full((1, 128)), _full((1, 128))],
        out_specs=_rows(_BE, 128),
        out_shape=jax.ShapeDtypeStruct((e, 128), F32),
    )(sp, dp, w1, b1, w2, b2, w3, b3, lns, lnb)


def _proj2(x, wsd):
    n = x.shape[0]

    def body(x_ref, w_ref, o_ref):
        o_ref[...] = _mm(x_ref[...], w_ref[...])

    o = pl.pallas_call(
        body,
        grid=(n // _BN,),
        in_specs=[_rows(_BN, 128), _full((128, 256))],
        out_specs=_rows(_BN, 256),
        out_shape=jax.ShapeDtypeStruct((n, 256), F32),
    )(x, wsd)
    return o[:, :128], o[:, 128:]


def _edge_step(he, gs, gd, w1e, b1, w2, b2, w3, b3, lns, lnb):
    e = he.shape[0]

    def body(he_ref, gs_ref, gd_ref, w1e_ref, b1_ref, w2_ref, b2_ref, w3_ref,
             b3_ref, s_ref, bb_ref, o_ref):
        he_v = he_ref[...]
        t = _mm(he_v, w1e_ref[...]) + gs_ref[...] + gd_ref[...] + b1_ref[...]
        t = jnp.maximum(t, 0.0)
        t = jnp.maximum(_mm(t, w2_ref[...]) + b2_ref[...], 0.0)
        t = _mm(t, w3_ref[...]) + b3_ref[...]
        o_ref[...] = he_v + _ln(t, s_ref[...], bb_ref[...])

    return pl.pallas_call(
        body,
        grid=(e // _BE,),
        in_specs=[_rows(_BE, 128), _rows(_BE, 128), _rows(_BE, 128),
                  _full((128, 128)), _full((1, 128)), _full((128, 128)),
                  _full((1, 128)), _full((128, 128)), _full((1, 128)),
                  _full((1, 128)), _full((1, 128))],
        out_specs=_rows(_BE, 128),
        out_shape=jax.ShapeDtypeStruct((e, 128), F32),
    )(he, gs, gd, w1e, b1, w2, b2, w3, b3, lns, lnb)


def _node_step(hn, agg, w1, b1, w2, b2, w3, b3, lns, lnb):
    n = hn.shape[0]

    def body(hn_ref, agg_ref, w1_ref, b1_ref, w2_ref, b2_ref, w3_ref,
             b3_ref, s_ref, bb_ref, o_ref):
        hn_v = hn_ref[...]
        x = jnp.concatenate((hn_v, agg_ref[...]), axis=1)
        t = _mm(x, w1_ref[...]) + b1_ref[...]
        t = jnp.maximum(t, 0.0)
        t = jnp.maximum(_mm(t, w2_ref[...]) + b2_ref[...], 0.0)
        t = _mm(t, w3_ref[...]) + b3_ref[...]
        o_ref[...] = hn_v + _ln(t, s_ref[...], bb_ref[...])

    return pl.pallas_call(
        body,
        grid=(n // _BN,),
        in_specs=[_rows(_BN, 128), _rows(_BN, 128),
                  _full((256, 128)), _full((1, 128)),
                  _full((128, 128)), _full((1, 128)), _full((128, 128)),
                  _full((1, 128)), _full((1, 128)), _full((1, 128))],
        out_specs=_rows(_BN, 128),
        out_shape=jax.ShapeDtypeStruct((n, 128), F32),
    )(hn, agg, w1, b1, w2, b2, w3, b3, lns, lnb)


def _decoder3(hn, w1, b1, w2, b2, w3, b3):
    n = hn.shape[0]

    def body(x_ref, w1_ref, b1_ref, w2_ref, b2_ref, w3_ref, b3_ref, o_ref):
        t = jnp.maximum(_mm(x_ref[...], w1_ref[...]) + b1_ref[...], 0.0)
        t = jnp.maximum(_mm(t, w2_ref[...]) + b2_ref[...], 0.0)
        o_ref[...] = _mm(t, w3_ref[...]) + b3_ref[...]

    return pl.pallas_call(
        body,
        grid=(n // _BN,),
        in_specs=[_rows(_BN, 128), _full((128, 128)), _full((1, 128)),
                  _full((128, 128)), _full((1, 128)), _full((128, 2)),
                  _full((1, 2))],
        out_specs=_rows(_BN, 2),
        out_shape=jax.ShapeDtypeStruct((n, 2), F32),
    )(hn, w1, b1, w2, b2, w3, b3)


def kernel(velocity, node_type, cells, mesh_pos, params):
    p = params
    n = velocity.shape[0]
    c0, c1, c2 = cells[:, 0], cells[:, 1], cells[:, 2]
    srcs = jnp.concatenate([c0, c1, c2, c1, c2, c0])
    dsts = jnp.concatenate([c1, c2, c0, c0, c1, c2])
    order = jnp.argsort(dsts)
    srcs = srcs[order].astype(jnp.int32)
    dsts = dsts[order].astype(jnp.int32)

    def r2(b):
        return b.reshape(1, -1)

    nmean, nstd = p['node_norm_mean'], p['node_norm_std']
    (w1n, b1n), (w2n, b2n), (w3n, b3n) = p['node_enc']
    w1n_f = w1n / nstd[:, None]
    b1n_f = b1n - (nmean / nstd) @ w1n
    w1v = w1n_f[:2]
    wtype = w1n_f[2:] + b1n_f[None, :]
    lns_n, lnb_n = p['node_enc_ln']
    h_n = _node_enc(velocity, node_type.reshape(-1, 1).astype(jnp.int32),
                    w1v, wtype, w2n, r2(b2n), w3n, r2(b3n), r2(lns_n), r2(lnb_n))

    emean, estd = p['edge_norm_mean'], p['edge_norm_std']
    (w1e, b1e), (w2e, b2e), (w3e, b3e) = p['edge_enc']
    w1e_f = w1e / estd[:, None]
    b1e_f = b1e - (emean / estd) @ w1e
    lns_e, lnb_e = p['edge_enc_ln']
    sp = jnp.take(mesh_pos, srcs, axis=0)
    dp = jnp.take(mesh_pos, dsts, axis=0)
    h_e = _edge_enc(sp, dp, w1e_f, r2(b1e_f), w2e, r2(b2e), w3e, r2(b3e),
                    r2(lns_e), r2(lnb_e))

    def stack(getter):
        return jnp.stack([getter(i) for i in range(len(p['mp_edge']))])

    xs = {
        'ew1e': stack(lambda i: p['mp_edge'][i][0][0][:128]),
        'ew1sd': stack(lambda i: jnp.concatenate(
            [p['mp_edge'][i][0][0][128:256], p['mp_edge'][i][0][0][256:]], axis=1)),
        'eb1': stack(lambda i: r2(p['mp_edge'][i][0][1])),
        'ew2': stack(lambda i: p['mp_edge'][i][1][0]),
        'eb2': stack(lambda i: r2(p['mp_edge'][i][1][1])),
        'ew3': stack(lambda i: p['mp_edge'][i][2][0]),
        'eb3': stack(lambda i: r2(p['mp_edge'][i][2][1])),
        'elns': stack(lambda i: r2(p['mp_edge_ln'][i][0])),
        'elnb': stack(lambda i: r2(p['mp_edge_ln'][i][1])),
        'nw1': stack(lambda i: p['mp_node'][i][0][0]),
        'nb1': stack(lambda i: r2(p['mp_node'][i][0][1])),
        'nw2': stack(lambda i: p['mp_node'][i][1][0]),
        'nb2': stack(lambda i: r2(p['mp_node'][i][1][1])),
        'nw3': stack(lambda i: p['mp_node'][i][2][0]),
        'nb3': stack(lambda i: r2(p['mp_node'][i][2][1])),
        'nlns': stack(lambda i: r2(p['mp_node_ln'][i][0])),
        'nlnb': stack(lambda i: r2(p['mp_node_ln'][i][1])),
    }

    def step(carry, w):
        h_n, h_e = carry
        gsf, gdf = _proj2(h_n, w['ew1sd'])
        gs = jnp.take(gsf, srcs, axis=0)
        gd = jnp.take(gdf, dsts, axis=0)
        h_e = _edge_step(h_e, gs, gd, w['ew1e'], w['eb1'], w['ew2'], w['eb2'],
                         w['ew3'], w['eb3'], w['elns'], w['elnb'])
        agg = jax.ops.segment_sum(h_e, dsts, num_segments=n)
        h_n = _node_step(h_n, agg, w['nw1'], w['nb1'], w['nw2'],
                         w['nb2'], w['nw3'], w['nb3'], w['nlns'], w['nlnb'])
        return (h_n, h_e), None

    (h_n, h_e), _ = lax.scan(step, (h_n, h_e), xs)

    (w1d, b1d), (w2d, b2d), (w3d, b3d) = p['decoder']
    w3d_f = w3d * p['out_norm_std'][None, :]
    b3d_f = b3d * p['out_norm_std'] + p['out_norm_mean']
    return _decoder3(h_n, w1d, r2(b1d), w2d, r2(b2d), w3d_f, r2(b3d_f))


# speedup vs baseline: 1.1333x; 1.1333x over previous
import jax
import jax.numpy as jnp
from jax import lax
from jax.experimental import pallas as pl
from jax.experimental.pallas import tpu as pltpu

F32 = jnp.float32
_BE = 2000
_BN = 2000


def _ln(x, s, b):
    mu = jnp.mean(x, axis=-1, keepdims=True)
    xc = x - mu
    var = jnp.mean(xc * xc, axis=-1, keepdims=True)
    return xc * lax.rsqrt(var + 1e-5) * s + b


def _mm(x, w):
    return jnp.dot(x, w, preferred_element_type=F32,
                   precision=lax.Precision.HIGHEST)


def _full(shape):
    return pl.BlockSpec(shape, lambda i: (0,) * len(shape))


def _rows(bs, w):
    return pl.BlockSpec((bs, w), lambda i: (i, 0))


def _node_enc(vel, ntype, w1v, wtype, w2, b2, w3, b3, lns, lnb):
    n = vel.shape[0]

    def body(vel_ref, t_ref, w1v_ref, wt_ref, w2_ref, b2_ref, w3_ref, b3_ref,
             s_ref, b_ref, o_ref):
        v = vel_ref[...]
        t = v[:, 0:1] * w1v_ref[0:1, :] + v[:, 1:2] * w1v_ref[1:2, :]
        tt = t_ref[...]
        for k in range(9):
            t = t + jnp.where(tt == k, 1.0, 0.0) * wt_ref[k:k + 1, :]
        t = jnp.maximum(t, 0.0)
        t = jnp.maximum(_mm(t, w2_ref[...]) + b2_ref[...], 0.0)
        t = _mm(t, w3_ref[...]) + b3_ref[...]
        o_ref[...] = _ln(t, s_ref[...], b_ref[...])

    return pl.pallas_call(
        body,
        grid=(n // _BN,),
        in_specs=[_rows(_BN, 2), _rows(_BN, 1), _full((2, 128)), _full((9, 128)),
                  _full((128, 128)), _full((1, 128)), _full((128, 128)),
                  _full((1, 128)), _full((1, 128)), _full((1, 128))],
        out_specs=_rows(_BN, 128),
        out_shape=jax.ShapeDtypeStruct((n, 128), F32),
    )(vel, ntype, w1v, wtype, w2, b2, w3, b3, lns, lnb)


def _edge_enc(sp, dp, w1, b1, w2, b2, w3, b3, lns, lnb):
    e = sp.shape[0]

    def body(sp_ref, dp_ref, w1_ref, b1_ref, w2_ref, b2_ref, w3_ref, b3_ref,
             s_ref, bb_ref, o_ref):
        r = sp_ref[...] - dp_ref[...]
        rx = r[:, 0:1]
        ry = r[:, 1:2]
        rn = jnp.sqrt(rx * rx + ry * ry)
        t = rx * w1_ref[0:1, :] + ry * w1_ref[1:2, :] + rn * w1_ref[2:3, :] + b1_ref[...]
        t = jnp.maximum(t, 0.0)
        t = jnp.maximum(_mm(t, w2_ref[...]) + b2_ref[...], 0.0)
        t = _mm(t, w3_ref[...]) + b3_ref[...]
        o_ref[...] = _ln(t, s_ref[...], bb_ref[...])

    return pl.pallas_call(
        body,
        grid=(e // _BE,),
        in_specs=[_rows(_BE, 2), _rows(_BE, 2), _full((3, 128)), _full((1, 128)),
                  _full((128, 128)), _full((1, 128)), _full((128, 128)),
                  _full((1, 128)), _full((1, 128)), _full((1, 128))],
        out_specs=_rows(_BE, 128),
        out_shape=jax.ShapeDtypeStruct((e, 128), F32),
    )(sp, dp, w1, b1, w2, b2, w3, b3, lns, lnb)


def _proj2(x, wsd):
    n = x.shape[0]

    def body(x_ref, w_ref, o_ref):
        o_ref[...] = _mm(x_ref[...], w_ref[...])

    o = pl.pallas_call(
        body,
        grid=(n // _BN,),
        in_specs=[_rows(_BN, 128), _full((128, 256))],
        out_specs=_rows(_BN, 256),
        out_shape=jax.ShapeDtypeStruct((n, 256), F32),
    )(x, wsd)
    return o[:, :128], o[:, 128:]


def _edge_step(he, gs, gd, w1e, b1, w2, b2, w3, b3, lns, lnb):
    e = he.shape[0]

    def body(he_ref, gs_ref, gd_ref, w1e_ref, b1_ref, w2_ref, b2_ref, w3_ref,
             b3_ref, s_ref, bb_ref, o_ref):
        he_v = he_ref[...]
        t = _mm(he_v, w1e_ref[...]) + gs_ref[...] + gd_ref[...] + b1_ref[...]
        t = jnp.maximum(t, 0.0)
        t = jnp.maximum(_mm(t, w2_ref[...]) + b2_ref[...], 0.0)
        t = _mm(t, w3_ref[...]) + b3_ref[...]
        o_ref[...] = he_v + _ln(t, s_ref[...], bb_ref[...])

    return pl.pallas_call(
        body,
        grid=(e // _BE,),
        in_specs=[_rows(_BE, 128), _rows(_BE, 128), _rows(_BE, 128),
                  _full((128, 128)), _full((1, 128)), _full((128, 128)),
                  _full((1, 128)), _full((128, 128)), _full((1, 128)),
                  _full((1, 128)), _full((1, 128))],
        out_specs=_rows(_BE, 128),
        out_shape=jax.ShapeDtypeStruct((e, 128), F32),
    )(he, gs, gd, w1e, b1, w2, b2, w3, b3, lns, lnb)


def _node_step(hn, agg, w1, b1, w2, b2, w3, b3, lns, lnb):
    n = hn.shape[0]

    def body(hn_ref, agg_ref, w1_ref, b1_ref, w2_ref, b2_ref, w3_ref,
             b3_ref, s_ref, bb_ref, o_ref):
        hn_v = hn_ref[...]
        x = jnp.concatenate((hn_v, agg_ref[...]), axis=1)
        t = _mm(x, w1_ref[...]) + b1_ref[...]
        t = jnp.maximum(t, 0.0)
        t = jnp.maximum(_mm(t, w2_ref[...]) + b2_ref[...], 0.0)
        t = _mm(t, w3_ref[...]) + b3_ref[...]
        o_ref[...] = hn_v + _ln(t, s_ref[...], bb_ref[...])

    return pl.pallas_call(
        body,
        grid=(n // _BN,),
        in_specs=[_rows(_BN, 128), _rows(_BN, 128),
                  _full((256, 128)), _full((1, 128)),
                  _full((128, 128)), _full((1, 128)), _full((128, 128)),
                  _full((1, 128)), _full((1, 128)), _full((1, 128))],
        out_specs=_rows(_BN, 128),
        out_shape=jax.ShapeDtypeStruct((n, 128), F32),
    )(hn, agg, w1, b1, w2, b2, w3, b3, lns, lnb)


def _decoder3(hn, w1, b1, w2, b2, w3, b3):
    n = hn.shape[0]

    def body(x_ref, w1_ref, b1_ref, w2_ref, b2_ref, w3_ref, b3_ref, o_ref):
        t = jnp.maximum(_mm(x_ref[...], w1_ref[...]) + b1_ref[...], 0.0)
        t = jnp.maximum(_mm(t, w2_ref[...]) + b2_ref[...], 0.0)
        o_ref[...] = _mm(t, w3_ref[...]) + b3_ref[...]

    return pl.pallas_call(
        body,
        grid=(n // _BN,),
        in_specs=[_rows(_BN, 128), _full((128, 128)), _full((1, 128)),
                  _full((128, 128)), _full((1, 128)), _full((128, 2)),
                  _full((1, 2))],
        out_specs=_rows(_BN, 2),
        out_shape=jax.ShapeDtypeStruct((n, 2), F32),
    )(hn, w1, b1, w2, b2, w3, b3)


def kernel(velocity, node_type, cells, mesh_pos, params):
    p = params
    n = velocity.shape[0]
    c0, c1, c2 = cells[:, 0], cells[:, 1], cells[:, 2]
    srcs = jnp.concatenate([c0, c1, c2, c1, c2, c0]).astype(jnp.int32)
    dsts = jnp.concatenate([c1, c2, c0, c0, c1, c2]).astype(jnp.int32)

    def r2(b):
        return b.reshape(1, -1)

    nmean, nstd = p['node_norm_mean'], p['node_norm_std']
    (w1n, b1n), (w2n, b2n), (w3n, b3n) = p['node_enc']
    w1n_f = w1n / nstd[:, None]
    b1n_f = b1n - (nmean / nstd) @ w1n
    w1v = w1n_f[:2]
    wtype = w1n_f[2:] + b1n_f[None, :]
    lns_n, lnb_n = p['node_enc_ln']
    h_n = _node_enc(velocity, node_type.reshape(-1, 1).astype(jnp.int32),
                    w1v, wtype, w2n, r2(b2n), w3n, r2(b3n), r2(lns_n), r2(lnb_n))

    emean, estd = p['edge_norm_mean'], p['edge_norm_std']
    (w1e, b1e), (w2e, b2e), (w3e, b3e) = p['edge_enc']
    w1e_f = w1e / estd[:, None]
    b1e_f = b1e - (emean / estd) @ w1e
    lns_e, lnb_e = p['edge_enc_ln']
    sp = jnp.take(mesh_pos, srcs, axis=0)
    dp = jnp.take(mesh_pos, dsts, axis=0)
    h_e = _edge_enc(sp, dp, w1e_f, r2(b1e_f), w2e, r2(b2e), w3e, r2(b3e),
                    r2(lns_e), r2(lnb_e))

    for i in range(len(p['mp_edge'])):
        ew1 = p['mp_edge'][i][0][0]
        ew1sd = jnp.concatenate([ew1[128:256], ew1[256:]], axis=1)
        gsf, gdf = _proj2(h_n, ew1sd)
        gs = jnp.take(gsf, srcs, axis=0)
        gd = jnp.take(gdf, dsts, axis=0)
        h_e = _edge_step(h_e, gs, gd, ew1[:128], r2(p['mp_edge'][i][0][1]),
                         p['mp_edge'][i][1][0], r2(p['mp_edge'][i][1][1]),
                         p['mp_edge'][i][2][0], r2(p['mp_edge'][i][2][1]),
                         r2(p['mp_edge_ln'][i][0]), r2(p['mp_edge_ln'][i][1]))
        agg = jax.ops.segment_sum(h_e, dsts, num_segments=n)
        h_n = _node_step(h_n, agg, p['mp_node'][i][0][0],
                         r2(p['mp_node'][i][0][1]),
                         p['mp_node'][i][1][0], r2(p['mp_node'][i][1][1]),
                         p['mp_node'][i][2][0], r2(p['mp_node'][i][2][1]),
                         r2(p['mp_node_ln'][i][0]), r2(p['mp_node_ln'][i][1]))

    (w1d, b1d), (w2d, b2d), (w3d, b3d) = p['decoder']
    w3d_f = w3d * p['out_norm_std'][None, :]
    b3d_f = b3d * p['out_norm_std'] + p['out_norm_mean']
    return _decoder3(h_n, w1d, r2(b1d), w2d, r2(b2d), w3d_f, r2(b3d_f))


# speedup vs baseline: 1.9971x; 1.7622x over previous
import functools

import jax
import jax.numpy as jnp
from jax import lax
from jax.experimental import pallas as pl
from jax.experimental.pallas import tpu as pltpu
from jax.experimental.pallas import tpu_sc as plsc

F32 = jnp.float32
_BE = 2000
_BN = 2000


def _ln(x, s, b):
    mu = jnp.mean(x, axis=-1, keepdims=True)
    xc = x - mu
    var = jnp.mean(xc * xc, axis=-1, keepdims=True)
    return xc * lax.rsqrt(var + 1e-5) * s + b


def _mm(x, w):
    return jnp.dot(x, w, preferred_element_type=F32,
                   precision=lax.Precision.HIGHEST)


def _full(shape):
    return pl.BlockSpec(shape, lambda i: (0,) * len(shape))


def _rows(bs, w):
    return pl.BlockSpec((bs, w), lambda i: (i, 0))


def _node_enc(vel, ntype, w1v, wtype, w2, b2, w3, b3, lns, lnb):
    n = vel.shape[0]

    def body(vel_ref, t_ref, w1v_ref, wt_ref, w2_ref, b2_ref, w3_ref, b3_ref,
             s_ref, b_ref, o_ref):
        v = vel_ref[...]
        t = v[:, 0:1] * w1v_ref[0:1, :] + v[:, 1:2] * w1v_ref[1:2, :]
        tt = t_ref[...]
        for k in range(9):
            t = t + jnp.where(tt == k, 1.0, 0.0) * wt_ref[k:k + 1, :]
        t = jnp.maximum(t, 0.0)
        t = jnp.maximum(_mm(t, w2_ref[...]) + b2_ref[...], 0.0)
        t = _mm(t, w3_ref[...]) + b3_ref[...]
        o_ref[...] = _ln(t, s_ref[...], b_ref[...])

    return pl.pallas_call(
        body,
        grid=(n // _BN,),
        in_specs=[_rows(_BN, 2), _rows(_BN, 1), _full((2, 128)), _full((9, 128)),
                  _full((128, 128)), _full((1, 128)), _full((128, 128)),
                  _full((1, 128)), _full((1, 128)), _full((1, 128))],
        out_specs=_rows(_BN, 128),
        out_shape=jax.ShapeDtypeStruct((n, 128), F32),
    )(vel, ntype, w1v, wtype, w2, b2, w3, b3, lns, lnb)


def _edge_enc(sp, dp, w1, b1, w2, b2, w3, b3, lns, lnb):
    e = sp.shape[0]

    def body(sp_ref, dp_ref, w1_ref, b1_ref, w2_ref, b2_ref, w3_ref, b3_ref,
             s_ref, bb_ref, o_ref):
        r = sp_ref[...] - dp_ref[...]
        rx = r[:, 0:1]
        ry = r[:, 1:2]
        rn = jnp.sqrt(rx * rx + ry * ry)
        t = rx * w1_ref[0:1, :] + ry * w1_ref[1:2, :] + rn * w1_ref[2:3, :] + b1_ref[...]
        t = jnp.maximum(t, 0.0)
        t = jnp.maximum(_mm(t, w2_ref[...]) + b2_ref[...], 0.0)
        t = _mm(t, w3_ref[...]) + b3_ref[...]
        o_ref[...] = _ln(t, s_ref[...], bb_ref[...])

    return pl.pallas_call(
        body,
        grid=(e // _BE,),
        in_specs=[_rows(_BE, 2), _rows(_BE, 2), _full((3, 128)), _full((1, 128)),
                  _full((128, 128)), _full((1, 128)), _full((128, 128)),
                  _full((1, 128)), _full((1, 128)), _full((1, 128))],
        out_specs=_rows(_BE, 128),
        out_shape=jax.ShapeDtypeStruct((e, 128), F32),
    )(sp, dp, w1, b1, w2, b2, w3, b3, lns, lnb)


def _proj2(x, wsd):
    n = x.shape[0]

    def body(x_ref, w_ref, os_ref, od_ref):
        t = _mm(x_ref[...], w_ref[...])
        os_ref[...] = t[:, :128]
        od_ref[...] = t[:, 128:]

    return pl.pallas_call(
        body,
        grid=(n // _BN,),
        in_specs=[_rows(_BN, 128), _full((128, 256))],
        out_specs=[_rows(_BN, 128), _rows(_BN, 128)],
        out_shape=[jax.ShapeDtypeStruct((n, 128), F32)] * 2,
    )(x, wsd)


_CH = 125
_NW = 32


def _sc_gather2(gsf, gdf, srcs2, dsts2):
    nchunks = srcs2.shape[0]
    per_w = nchunks // _NW
    mesh = plsc.VectorSubcoreMesh(core_axis_name="c", subcore_axis_name="s")

    @functools.partial(
        pl.kernel, mesh=mesh,
        out_type=[jax.ShapeDtypeStruct((nchunks, _CH, 128), F32)] * 2,
        scratch_types=[
            pltpu.VMEM((_CH,), jnp.int32),
            pltpu.VMEM((_CH,), jnp.int32),
            pltpu.VMEM((_CH, 128), F32),
            pltpu.VMEM((_CH, 128), F32),
            pltpu.SemaphoreType.DMA,
            pltpu.SemaphoreType.DMA,
        ],
    )
    def k(gsf_hbm, gdf_hbm, s_hbm, d_hbm, os_hbm, od_hbm,
          idx_s, idx_d, buf_s, buf_d, sem_s, sem_d):
        wid = lax.axis_index("s") * 2 + lax.axis_index("c")

        def body(j, carry):
            chunk = wid * per_w + j
            pltpu.sync_copy(s_hbm.at[chunk], idx_s)
            pltpu.sync_copy(d_hbm.at[chunk], idx_d)
            cs = pltpu.async_copy(gsf_hbm.at[idx_s], buf_s, sem_s)
            cd = pltpu.async_copy(gdf_hbm.at[idx_d], buf_d, sem_d)
            cs.wait()
            cd.wait()
            pltpu.sync_copy(buf_s, os_hbm.at[chunk])
            pltpu.sync_copy(buf_d, od_hbm.at[chunk])
            return carry

        lax.fori_loop(0, per_w, body, 0)

    return k(gsf, gdf, srcs2, dsts2)


def _edge_step(he, gs, gd, w1e, b1, w2, b2, w3, b3, lns, lnb):
    e = he.shape[0]

    def body(he_ref, gs_ref, gd_ref, w1e_ref, b1_ref, w2_ref, b2_ref, w3_ref,
             b3_ref, s_ref, bb_ref, o_ref):
        he_v = he_ref[...]
        t = _mm(he_v, w1e_ref[...]) + gs_ref[...] + gd_ref[...] + b1_ref[...]
        t = jnp.maximum(t, 0.0)
        t = jnp.maximum(_mm(t, w2_ref[...]) + b2_ref[...], 0.0)
        t = _mm(t, w3_ref[...]) + b3_ref[...]
        o_ref[...] = he_v + _ln(t, s_ref[...], bb_ref[...])

    return pl.pallas_call(
        body,
        grid=(e // _BE,),
        in_specs=[_rows(_BE, 128), _rows(_BE, 128), _rows(_BE, 128),
                  _full((128, 128)), _full((1, 128)), _full((128, 128)),
                  _full((1, 128)), _full((128, 128)), _full((1, 128)),
                  _full((1, 128)), _full((1, 128))],
        out_specs=_rows(_BE, 128),
        out_shape=jax.ShapeDtypeStruct((e, 128), F32),
    )(he, gs, gd, w1e, b1, w2, b2, w3, b3, lns, lnb)


def _node_step(hn, agg, w1, b1, w2, b2, w3, b3, lns, lnb):
    n = hn.shape[0]

    def body(hn_ref, agg_ref, w1_ref, b1_ref, w2_ref, b2_ref, w3_ref,
             b3_ref, s_ref, bb_ref, o_ref):
        hn_v = hn_ref[...]
        x = jnp.concatenate((hn_v, agg_ref[...]), axis=1)
        t = _mm(x, w1_ref[...]) + b1_ref[...]
        t = jnp.maximum(t, 0.0)
        t = jnp.maximum(_mm(t, w2_ref[...]) + b2_ref[...], 0.0)
        t = _mm(t, w3_ref[...]) + b3_ref[...]
        o_ref[...] = hn_v + _ln(t, s_ref[...], bb_ref[...])

    return pl.pallas_call(
        body,
        grid=(n // _BN,),
        in_specs=[_rows(_BN, 128), _rows(_BN, 128),
                  _full((256, 128)), _full((1, 128)),
                  _full((128, 128)), _full((1, 128)), _full((128, 128)),
                  _full((1, 128)), _full((1, 128)), _full((1, 128))],
        out_specs=_rows(_BN, 128),
        out_shape=jax.ShapeDtypeStruct((n, 128), F32),
    )(hn, agg, w1, b1, w2, b2, w3, b3, lns, lnb)


def _decoder3(hn, w1, b1, w2, b2, w3, b3):
    n = hn.shape[0]

    def body(x_ref, w1_ref, b1_ref, w2_ref, b2_ref, w3_ref, b3_ref, o_ref):
        t = jnp.maximum(_mm(x_ref[...], w1_ref[...]) + b1_ref[...], 0.0)
        t = jnp.maximum(_mm(t, w2_ref[...]) + b2_ref[...], 0.0)
        o_ref[...] = _mm(t, w3_ref[...]) + b3_ref[...]

    return pl.pallas_call(
        body,
        grid=(n // _BN,),
        in_specs=[_rows(_BN, 128), _full((128, 128)), _full((1, 128)),
                  _full((128, 128)), _full((1, 128)), _full((128, 2)),
                  _full((1, 2))],
        out_specs=_rows(_BN, 2),
        out_shape=jax.ShapeDtypeStruct((n, 2), F32),
    )(hn, w1, b1, w2, b2, w3, b3)


def kernel(velocity, node_type, cells, mesh_pos, params):
    p = params
    n = velocity.shape[0]
    c0, c1, c2 = cells[:, 0], cells[:, 1], cells[:, 2]
    srcs = jnp.concatenate([c0, c1, c2, c1, c2, c0]).astype(jnp.int32)
    dsts = jnp.concatenate([c1, c2, c0, c0, c1, c2]).astype(jnp.int32)

    def r2(b):
        return b.reshape(1, -1)

    nmean, nstd = p['node_norm_mean'], p['node_norm_std']
    (w1n, b1n), (w2n, b2n), (w3n, b3n) = p['node_enc']
    w1n_f = w1n / nstd[:, None]
    b1n_f = b1n - (nmean / nstd) @ w1n
    w1v = w1n_f[:2]
    wtype = w1n_f[2:] + b1n_f[None, :]
    lns_n, lnb_n = p['node_enc_ln']
    h_n = _node_enc(velocity, node_type.reshape(-1, 1).astype(jnp.int32),
                    w1v, wtype, w2n, r2(b2n), w3n, r2(b3n), r2(lns_n), r2(lnb_n))

    emean, estd = p['edge_norm_mean'], p['edge_norm_std']
    (w1e, b1e), (w2e, b2e), (w3e, b3e) = p['edge_enc']
    w1e_f = w1e / estd[:, None]
    b1e_f = b1e - (emean / estd) @ w1e
    lns_e, lnb_e = p['edge_enc_ln']
    sp = jnp.take(mesh_pos, srcs, axis=0)
    dp = jnp.take(mesh_pos, dsts, axis=0)
    h_e = _edge_enc(sp, dp, w1e_f, r2(b1e_f), w2e, r2(b2e), w3e, r2(b3e),
                    r2(lns_e), r2(lnb_e))

    e = srcs.shape[0]
    srcs2 = srcs.reshape(-1, _CH)
    dsts2 = dsts.reshape(-1, _CH)
    for i in range(len(p['mp_edge'])):
        ew1 = p['mp_edge'][i][0][0]
        ew1sd = jnp.concatenate([ew1[128:256], ew1[256:]], axis=1)
        gsf, gdf = _proj2(h_n, ew1sd)
        gs3, gd3 = _sc_gather2(gsf, gdf, srcs2, dsts2)
        gs = gs3.reshape(e, 128)
        gd = gd3.reshape(e, 128)
        h_e = _edge_step(h_e, gs, gd, ew1[:128], r2(p['mp_edge'][i][0][1]),
                         p['mp_edge'][i][1][0], r2(p['mp_edge'][i][1][1]),
                         p['mp_edge'][i][2][0], r2(p['mp_edge'][i][2][1]),
                         r2(p['mp_edge_ln'][i][0]), r2(p['mp_edge_ln'][i][1]))
        agg = jax.ops.segment_sum(h_e, dsts, num_segments=n)
        h_n = _node_step(h_n, agg, p['mp_node'][i][0][0],
                         r2(p['mp_node'][i][0][1]),
                         p['mp_node'][i][1][0], r2(p['mp_node'][i][1][1]),
                         p['mp_node'][i][2][0], r2(p['mp_node'][i][2][1]),
                         r2(p['mp_node_ln'][i][0]), r2(p['mp_node_ln'][i][1]))

    (w1d, b1d), (w2d, b2d), (w3d, b3d) = p['decoder']
    w3d_f = w3d * p['out_norm_std'][None, :]
    b3d_f = b3d * p['out_norm_std'] + p['out_norm_mean']
    return _decoder3(h_n, w1d, r2(b1d), w2d, r2(b2d), w3d_f, r2(b3d_f))


# speedup vs baseline: 2.3506x; 1.1770x over previous
import functools

import jax
import jax.numpy as jnp
from jax import lax
from jax.experimental import pallas as pl
from jax.experimental.pallas import tpu as pltpu
from jax.experimental.pallas import tpu_sc as plsc

F32 = jnp.float32
_BE = 2000
_BN = 2000


def _ln(x, s, b):
    mu = jnp.mean(x, axis=-1, keepdims=True)
    xc = x - mu
    var = jnp.mean(xc * xc, axis=-1, keepdims=True)
    return xc * lax.rsqrt(var + 1e-5) * s + b


def _mm(x, w):
    return jnp.dot(x, w, preferred_element_type=F32,
                   precision=lax.Precision.HIGHEST)


def _full(shape):
    return pl.BlockSpec(shape, lambda i: (0,) * len(shape))


def _rows(bs, w):
    return pl.BlockSpec((bs, w), lambda i: (i, 0))


def _node_enc(vel, ntype, w1v, wtype, w2, b2, w3, b3, lns, lnb):
    n = vel.shape[0]

    def body(vel_ref, t_ref, w1v_ref, wt_ref, w2_ref, b2_ref, w3_ref, b3_ref,
             s_ref, b_ref, o_ref):
        v = vel_ref[...]
        t = v[:, 0:1] * w1v_ref[0:1, :] + v[:, 1:2] * w1v_ref[1:2, :]
        tt = t_ref[...]
        for k in range(9):
            t = t + jnp.where(tt == k, 1.0, 0.0) * wt_ref[k:k + 1, :]
        t = jnp.maximum(t, 0.0)
        t = jnp.maximum(_mm(t, w2_ref[...]) + b2_ref[...], 0.0)
        t = _mm(t, w3_ref[...]) + b3_ref[...]
        o_ref[...] = _ln(t, s_ref[...], b_ref[...])

    return pl.pallas_call(
        body,
        grid=(n // _BN,),
        in_specs=[_rows(_BN, 2), _rows(_BN, 1), _full((2, 128)), _full((9, 128)),
                  _full((128, 128)), _full((1, 128)), _full((128, 128)),
                  _full((1, 128)), _full((1, 128)), _full((1, 128))],
        out_specs=_rows(_BN, 128),
        out_shape=jax.ShapeDtypeStruct((n, 128), F32),
    )(vel, ntype, w1v, wtype, w2, b2, w3, b3, lns, lnb)


def _edge_enc(e, sp, dp, w1, b1, w2, b2, w3, b3, lns, lnb):

    def body(sp_ref, dp_ref, w1_ref, b1_ref, w2_ref, b2_ref, w3_ref, b3_ref,
             s_ref, bb_ref, o_ref):
        r = sp_ref[:, :2] - dp_ref[:, :2]
        rx = r[:, 0:1]
        ry = r[:, 1:2]
        rn = jnp.sqrt(rx * rx + ry * ry)
        t = rx * w1_ref[0:1, :] + ry * w1_ref[1:2, :] + rn * w1_ref[2:3, :] + b1_ref[...]
        t = jnp.maximum(t, 0.0)
        t = jnp.maximum(_mm(t, w2_ref[...]) + b2_ref[...], 0.0)
        t = _mm(t, w3_ref[...]) + b3_ref[...]
        o_ref[...] = _ln(t, s_ref[...], bb_ref[...])

    return pl.pallas_call(
        body,
        grid=(e // _BE,),
        in_specs=[_rows(_BE, 128), _rows(_BE, 128), _full((3, 128)), _full((1, 128)),
                  _full((128, 128)), _full((1, 128)), _full((128, 128)),
                  _full((1, 128)), _full((1, 128)), _full((1, 128))],
        out_specs=_rows(_BE, 128),
        out_shape=jax.ShapeDtypeStruct((e, 128), F32),
    )(sp, dp, w1, b1, w2, b2, w3, b3, lns, lnb)


def _proj2(x, wsd):
    n = x.shape[0]

    def body(x_ref, w_ref, os_ref, od_ref):
        t = _mm(x_ref[...], w_ref[...])
        os_ref[...] = t[:, :128]
        od_ref[...] = t[:, 128:]

    return pl.pallas_call(
        body,
        grid=(n // _BN,),
        in_specs=[_rows(_BN, 128), _full((128, 256))],
        out_specs=[_rows(_BN, 128), _rows(_BN, 128)],
        out_shape=[jax.ShapeDtypeStruct((n, 128), F32)] * 2,
    )(x, wsd)


_CH = 128
_NW = 32


def _sc_gather2(ta, tb, ia2, ib2, width):
    nchunks = ia2.shape[0]
    per_w = nchunks // _NW
    e = nchunks * _CH
    mesh = plsc.VectorSubcoreMesh(core_axis_name="c", subcore_axis_name="s")

    @functools.partial(
        pl.kernel, mesh=mesh,
        out_type=[jax.ShapeDtypeStruct((e, width), F32)] * 2,
        scratch_types=[
            pltpu.VMEM((_CH,), jnp.int32),
            pltpu.VMEM((_CH,), jnp.int32),
            pltpu.VMEM((_CH, width), F32),
            pltpu.VMEM((_CH, width), F32),
            pltpu.SemaphoreType.DMA,
            pltpu.SemaphoreType.DMA,
        ],
    )
    def k(ta_hbm, tb_hbm, s_hbm, d_hbm, oa_hbm, ob_hbm,
          idx_s, idx_d, buf_s, buf_d, sem_s, sem_d):
        wid = lax.axis_index("s") * 2 + lax.axis_index("c")

        def body(j, carry):
            chunk = wid * per_w + j
            base = chunk * _CH
            pltpu.sync_copy(s_hbm.at[chunk], idx_s)
            pltpu.sync_copy(d_hbm.at[chunk], idx_d)
            cs = pltpu.async_copy(ta_hbm.at[idx_s], buf_s, sem_s)
            cd = pltpu.async_copy(tb_hbm.at[idx_d], buf_d, sem_d)
            cs.wait()
            cd.wait()
            pltpu.sync_copy(buf_s, oa_hbm.at[pl.ds(base, _CH)])
            pltpu.sync_copy(buf_d, ob_hbm.at[pl.ds(base, _CH)])
            return carry

        lax.fori_loop(0, per_w, body, 0)

    return k(ta, tb, ia2, ib2)


def _edge_step(he, gs, gd, w1e, b1, w2, b2, w3, b3, lns, lnb):
    e = he.shape[0]

    def body(he_ref, gs_ref, gd_ref, w1e_ref, b1_ref, w2_ref, b2_ref, w3_ref,
             b3_ref, s_ref, bb_ref, o_ref):
        he_v = he_ref[...]
        t = _mm(he_v, w1e_ref[...]) + gs_ref[...] + gd_ref[...] + b1_ref[...]
        t = jnp.maximum(t, 0.0)
        t = jnp.maximum(_mm(t, w2_ref[...]) + b2_ref[...], 0.0)
        t = _mm(t, w3_ref[...]) + b3_ref[...]
        o_ref[...] = he_v + _ln(t, s_ref[...], bb_ref[...])

    return pl.pallas_call(
        body,
        grid=(e // _BE,),
        in_specs=[_rows(_BE, 128), _rows(_BE, 128), _rows(_BE, 128),
                  _full((128, 128)), _full((1, 128)), _full((128, 128)),
                  _full((1, 128)), _full((128, 128)), _full((1, 128)),
                  _full((1, 128)), _full((1, 128))],
        out_specs=_rows(_BE, 128),
        out_shape=jax.ShapeDtypeStruct((e, 128), F32),
    )(he, gs, gd, w1e, b1, w2, b2, w3, b3, lns, lnb)


def _node_step(hn, agg, w1, b1, w2, b2, w3, b3, lns, lnb):
    n = hn.shape[0]

    def body(hn_ref, agg_ref, w1_ref, b1_ref, w2_ref, b2_ref, w3_ref,
             b3_ref, s_ref, bb_ref, o_ref):
        hn_v = hn_ref[...]
        x = jnp.concatenate((hn_v, agg_ref[...]), axis=1)
        t = _mm(x, w1_ref[...]) + b1_ref[...]
        t = jnp.maximum(t, 0.0)
        t = jnp.maximum(_mm(t, w2_ref[...]) + b2_ref[...], 0.0)
        t = _mm(t, w3_ref[...]) + b3_ref[...]
        o_ref[...] = hn_v + _ln(t, s_ref[...], bb_ref[...])

    return pl.pallas_call(
        body,
        grid=(n // _BN,),
        in_specs=[_rows(_BN, 128), _rows(_BN, 128),
                  _full((256, 128)), _full((1, 128)),
                  _full((128, 128)), _full((1, 128)), _full((128, 128)),
                  _full((1, 128)), _full((1, 128)), _full((1, 128))],
        out_specs=_rows(_BN, 128),
        out_shape=jax.ShapeDtypeStruct((n, 128), F32),
    )(hn, agg, w1, b1, w2, b2, w3, b3, lns, lnb)


def _decoder3(hn, w1, b1, w2, b2, w3, b3):
    n = hn.shape[0]

    def body(x_ref, w1_ref, b1_ref, w2_ref, b2_ref, w3_ref, b3_ref, o_ref):
        t = jnp.maximum(_mm(x_ref[...], w1_ref[...]) + b1_ref[...], 0.0)
        t = jnp.maximum(_mm(t, w2_ref[...]) + b2_ref[...], 0.0)
        o_ref[...] = _mm(t, w3_ref[...]) + b3_ref[...]

    return pl.pallas_call(
        body,
        grid=(n // _BN,),
        in_specs=[_rows(_BN, 128), _full((128, 128)), _full((1, 128)),
                  _full((128, 128)), _full((1, 128)), _full((128, 2)),
                  _full((1, 2))],
        out_specs=_rows(_BN, 2),
        out_shape=jax.ShapeDtypeStruct((n, 2), F32),
    )(hn, w1, b1, w2, b2, w3, b3)


def kernel(velocity, node_type, cells, mesh_pos, params):
    p = params
    n = velocity.shape[0]
    c0, c1, c2 = cells[:, 0], cells[:, 1], cells[:, 2]
    srcs = jnp.concatenate([c0, c1, c2, c1, c2, c0]).astype(jnp.int32)
    dsts = jnp.concatenate([c1, c2, c0, c0, c1, c2]).astype(jnp.int32)

    def r2(b):
        return b.reshape(1, -1)

    nmean, nstd = p['node_norm_mean'], p['node_norm_std']
    (w1n, b1n), (w2n, b2n), (w3n, b3n) = p['node_enc']
    w1n_f = w1n / nstd[:, None]
    b1n_f = b1n - (nmean / nstd) @ w1n
    w1v = w1n_f[:2]
    wtype = w1n_f[2:] + b1n_f[None, :]
    lns_n, lnb_n = p['node_enc_ln']
    h_n = _node_enc(velocity, node_type.reshape(-1, 1).astype(jnp.int32),
                    w1v, wtype, w2n, r2(b2n), w3n, r2(b3n), r2(lns_n), r2(lnb_n))

    emean, estd = p['edge_norm_mean'], p['edge_norm_std']
    (w1e, b1e), (w2e, b2e), (w3e, b3e) = p['edge_enc']
    w1e_f = w1e / estd[:, None]
    b1e_f = b1e - (emean / estd) @ w1e
    lns_e, lnb_e = p['edge_enc_ln']
    epad = (-srcs.shape[0]) % (_CH * _NW)
    srcs2 = jnp.pad(srcs, (0, epad)).reshape(-1, _CH)
    dsts2 = jnp.pad(dsts, (0, epad)).reshape(-1, _CH)
    mp128 = jnp.pad(mesh_pos, ((0, 0), (0, 126)))
    sp, dp = _sc_gather2(mp128, mp128, srcs2, dsts2, 128)
    h_e = _edge_enc(srcs.shape[0], sp, dp, w1e_f, r2(b1e_f), w2e, r2(b2e),
                    w3e, r2(b3e), r2(lns_e), r2(lnb_e))

    for i in range(len(p['mp_edge'])):
        ew1 = p['mp_edge'][i][0][0]
        ew1sd = jnp.concatenate([ew1[128:256], ew1[256:]], axis=1)
        gsf, gdf = _proj2(h_n, ew1sd)
        gs, gd = _sc_gather2(gsf, gdf, srcs2, dsts2, 128)
        h_e = _edge_step(h_e, gs, gd, ew1[:128], r2(p['mp_edge'][i][0][1]),
                         p['mp_edge'][i][1][0], r2(p['mp_edge'][i][1][1]),
                         p['mp_edge'][i][2][0], r2(p['mp_edge'][i][2][1]),
                         r2(p['mp_edge_ln'][i][0]), r2(p['mp_edge_ln'][i][1]))
        agg = jax.ops.segment_sum(h_e, dsts, num_segments=n)
        h_n = _node_step(h_n, agg, p['mp_node'][i][0][0],
                         r2(p['mp_node'][i][0][1]),
                         p['mp_node'][i][1][0], r2(p['mp_node'][i][1][1]),
                         p['mp_node'][i][2][0], r2(p['mp_node'][i][2][1]),
                         r2(p['mp_node_ln'][i][0]), r2(p['mp_node_ln'][i][1]))

    (w1d, b1d), (w2d, b2d), (w3d, b3d) = p['decoder']
    w3d_f = w3d * p['out_norm_std'][None, :]
    b3d_f = b3d * p['out_norm_std'] + p['out_norm_mean']
    return _decoder3(h_n, w1d, r2(b1d), w2d, r2(b2d), w3d_f, r2(b3d_f))


# speedup vs baseline: 2.8587x; 1.2162x over previous
import functools

import jax
import jax.numpy as jnp
from jax import lax
from jax.experimental import pallas as pl
from jax.experimental.pallas import tpu as pltpu
from jax.experimental.pallas import tpu_sc as plsc

F32 = jnp.float32
_BE = 2000
_BN = 2000


def _ln(x, s, b):
    mu = jnp.mean(x, axis=-1, keepdims=True)
    xc = x - mu
    var = jnp.mean(xc * xc, axis=-1, keepdims=True)
    return xc * lax.rsqrt(var + 1e-5) * s + b


def _mm(x, w):
    return jnp.dot(x, w, preferred_element_type=F32,
                   precision=lax.Precision.HIGHEST)


def _full(shape):
    return pl.BlockSpec(shape, lambda i: (0,) * len(shape))


def _rows(bs, w):
    return pl.BlockSpec((bs, w), lambda i: (i, 0))


def _node_enc(vel, ntype, w1v, wtype, w2, b2, w3, b3, lns, lnb):
    n = vel.shape[0]

    def body(vel_ref, t_ref, w1v_ref, wt_ref, w2_ref, b2_ref, w3_ref, b3_ref,
             s_ref, b_ref, o_ref):
        v = vel_ref[...]
        t = v[:, 0:1] * w1v_ref[0:1, :] + v[:, 1:2] * w1v_ref[1:2, :]
        tt = t_ref[...]
        for k in range(9):
            t = t + jnp.where(tt == k, 1.0, 0.0) * wt_ref[k:k + 1, :]
        t = jnp.maximum(t, 0.0)
        t = jnp.maximum(_mm(t, w2_ref[...]) + b2_ref[...], 0.0)
        t = _mm(t, w3_ref[...]) + b3_ref[...]
        o_ref[...] = _ln(t, s_ref[...], b_ref[...])

    return pl.pallas_call(
        body,
        grid=(n // _BN,),
        in_specs=[_rows(_BN, 2), _rows(_BN, 1), _full((2, 128)), _full((9, 128)),
                  _full((128, 128)), _full((1, 128)), _full((128, 128)),
                  _full((1, 128)), _full((1, 128)), _full((1, 128))],
        out_specs=_rows(_BN, 128),
        out_shape=jax.ShapeDtypeStruct((n, 128), F32),
    )(vel, ntype, w1v, wtype, w2, b2, w3, b3, lns, lnb)


def _edge_enc(e, sp, dp, w1, b1, w2, b2, w3, b3, lns, lnb):

    def body(sp_ref, dp_ref, w1_ref, b1_ref, w2_ref, b2_ref, w3_ref, b3_ref,
             s_ref, bb_ref, o_ref):
        r = sp_ref[:, :2] - dp_ref[:, :2]
        rx = r[:, 0:1]
        ry = r[:, 1:2]
        rn = jnp.sqrt(rx * rx + ry * ry)
        t = rx * w1_ref[0:1, :] + ry * w1_ref[1:2, :] + rn * w1_ref[2:3, :] + b1_ref[...]
        t = jnp.maximum(t, 0.0)
        t = jnp.maximum(_mm(t, w2_ref[...]) + b2_ref[...], 0.0)
        t = _mm(t, w3_ref[...]) + b3_ref[...]
        o_ref[...] = _ln(t, s_ref[...], bb_ref[...])

    return pl.pallas_call(
        body,
        grid=(e // _BE,),
        in_specs=[_rows(_BE, 128), _rows(_BE, 128), _full((3, 128)), _full((1, 128)),
                  _full((128, 128)), _full((1, 128)), _full((128, 128)),
                  _full((1, 128)), _full((1, 128)), _full((1, 128))],
        out_specs=_rows(_BE, 128),
        out_shape=jax.ShapeDtypeStruct((e, 128), F32),
    )(sp, dp, w1, b1, w2, b2, w3, b3, lns, lnb)


def _proj2(x, wsd):
    n = x.shape[0]

    def body(x_ref, w_ref, os_ref, od_ref):
        t = _mm(x_ref[...], w_ref[...])
        os_ref[...] = t[:, :128]
        od_ref[...] = t[:, 128:]

    return pl.pallas_call(
        body,
        grid=(n // _BN,),
        in_specs=[_rows(_BN, 128), _full((128, 256))],
        out_specs=[_rows(_BN, 128), _rows(_BN, 128)],
        out_shape=[jax.ShapeDtypeStruct((n, 128), F32)] * 2,
    )(x, wsd)


_CH = 128
_NW = 32


def _sc_gather2(ta, tb, ia2, ib2, width):
    nchunks = ia2.shape[0]
    per_w = nchunks // _NW
    e = nchunks * _CH
    mesh = plsc.VectorSubcoreMesh(core_axis_name="c", subcore_axis_name="s")

    @functools.partial(
        pl.kernel, mesh=mesh,
        out_type=[jax.ShapeDtypeStruct((e, width), F32)] * 2,
        scratch_types=[
            pltpu.VMEM((_CH,), jnp.int32),
            pltpu.VMEM((_CH,), jnp.int32),
            pltpu.VMEM((_CH, width), F32),
            pltpu.VMEM((_CH, width), F32),
            pltpu.SemaphoreType.DMA,
            pltpu.SemaphoreType.DMA,
        ],
    )
    def k(ta_hbm, tb_hbm, s_hbm, d_hbm, oa_hbm, ob_hbm,
          idx_s, idx_d, buf_s, buf_d, sem_s, sem_d):
        wid = lax.axis_index("s") * 2 + lax.axis_index("c")

        def body(j, carry):
            chunk = wid * per_w + j
            base = chunk * _CH
            pltpu.sync_copy(s_hbm.at[chunk], idx_s)
            pltpu.sync_copy(d_hbm.at[chunk], idx_d)
            cs = pltpu.async_copy(ta_hbm.at[idx_s], buf_s, sem_s)
            cd = pltpu.async_copy(tb_hbm.at[idx_d], buf_d, sem_d)
            cs.wait()
            cd.wait()
            pltpu.sync_copy(buf_s, oa_hbm.at[pl.ds(base, _CH)])
            pltpu.sync_copy(buf_d, ob_hbm.at[pl.ds(base, _CH)])
            return carry

        lax.fori_loop(0, per_w, body, 0)

    return k(ta, tb, ia2, ib2)


def _edge_step(he, gs, gd, w1e, b1, w2, b2, w3, b3, lns, lnb):
    e = he.shape[0]

    def body(he_ref, gs_ref, gd_ref, w1e_ref, b1_ref, w2_ref, b2_ref, w3_ref,
             b3_ref, s_ref, bb_ref, o_ref):
        he_v = he_ref[...]
        t = _mm(he_v, w1e_ref[...]) + gs_ref[...] + gd_ref[...] + b1_ref[...]
        t = jnp.maximum(t, 0.0)
        t = jnp.maximum(_mm(t, w2_ref[...]) + b2_ref[...], 0.0)
        t = _mm(t, w3_ref[...]) + b3_ref[...]
        o_ref[...] = he_v + _ln(t, s_ref[...], bb_ref[...])

    return pl.pallas_call(
        body,
        grid=(e // _BE,),
        in_specs=[_rows(_BE, 128), _rows(_BE, 128), _rows(_BE, 128),
                  _full((128, 128)), _full((1, 128)), _full((128, 128)),
                  _full((1, 128)), _full((128, 128)), _full((1, 128)),
                  _full((1, 128)), _full((1, 128))],
        out_specs=_rows(_BE, 128),
        out_shape=jax.ShapeDtypeStruct((e, 128), F32),
    )(he, gs, gd, w1e, b1, w2, b2, w3, b3, lns, lnb)


def _node_step(hn, agg, w1, b1, w2, b2, w3, b3, lns, lnb):
    n = hn.shape[0]

    def body(hn_ref, agg_ref, w1_ref, b1_ref, w2_ref, b2_ref, w3_ref,
             b3_ref, s_ref, bb_ref, o_ref):
        hn_v = hn_ref[...]
        x = jnp.concatenate((hn_v, agg_ref[...]), axis=1)
        t = _mm(x, w1_ref[...]) + b1_ref[...]
        t = jnp.maximum(t, 0.0)
        t = jnp.maximum(_mm(t, w2_ref[...]) + b2_ref[...], 0.0)
        t = _mm(t, w3_ref[...]) + b3_ref[...]
        o_ref[...] = hn_v + _ln(t, s_ref[...], bb_ref[...])

    return pl.pallas_call(
        body,
        grid=(n // _BN,),
        in_specs=[_rows(_BN, 128), _rows(_BN, 128),
                  _full((256, 128)), _full((1, 128)),
                  _full((128, 128)), _full((1, 128)), _full((128, 128)),
                  _full((1, 128)), _full((1, 128)), _full((1, 128))],
        out_specs=_rows(_BN, 128),
        out_shape=jax.ShapeDtypeStruct((n, 128), F32),
    )(hn, agg, w1, b1, w2, b2, w3, b3, lns, lnb)


def _decoder3(hn, w1, b1, w2, b2, w3, b3):
    n = hn.shape[0]

    def body(x_ref, w1_ref, b1_ref, w2_ref, b2_ref, w3_ref, b3_ref, o_ref):
        t = jnp.maximum(_mm(x_ref[...], w1_ref[...]) + b1_ref[...], 0.0)
        t = jnp.maximum(_mm(t, w2_ref[...]) + b2_ref[...], 0.0)
        o_ref[...] = _mm(t, w3_ref[...]) + b3_ref[...]

    return pl.pallas_call(
        body,
        grid=(n // _BN,),
        in_specs=[_rows(_BN, 128), _full((128, 128)), _full((1, 128)),
                  _full((128, 128)), _full((1, 128)), _full((128, 2)),
                  _full((1, 2))],
        out_specs=_rows(_BN, 2),
        out_shape=jax.ShapeDtypeStruct((n, 2), F32),
    )(hn, w1, b1, w2, b2, w3, b3)


def kernel(velocity, node_type, cells, mesh_pos, params):
    p = params
    n = velocity.shape[0]
    c0, c1, c2 = cells[:, 0], cells[:, 1], cells[:, 2]
    srcs = jnp.concatenate([c0, c1, c2, c1, c2, c0]).astype(jnp.int32)
    dsts = jnp.concatenate([c1, c2, c0, c0, c1, c2]).astype(jnp.int32)

    def r2(b):
        return b.reshape(1, -1)

    nmean, nstd = p['node_norm_mean'], p['node_norm_std']
    (w1n, b1n), (w2n, b2n), (w3n, b3n) = p['node_enc']
    w1n_f = w1n / nstd[:, None]
    b1n_f = b1n - (nmean / nstd) @ w1n
    w1v = w1n_f[:2]
    wtype = w1n_f[2:] + b1n_f[None, :]
    lns_n, lnb_n = p['node_enc_ln']
    h_n = _node_enc(velocity, node_type.reshape(-1, 1).astype(jnp.int32),
                    w1v, wtype, w2n, r2(b2n), w3n, r2(b3n), r2(lns_n), r2(lnb_n))

    emean, estd = p['edge_norm_mean'], p['edge_norm_std']
    (w1e, b1e), (w2e, b2e), (w3e, b3e) = p['edge_enc']
    w1e_f = w1e / estd[:, None]
    b1e_f = b1e - (emean / estd) @ w1e
    lns_e, lnb_e = p['edge_enc_ln']

    e_all = srcs.shape[0]
    eh = e_all // 2
    halves = []
    mp128 = jnp.pad(mesh_pos, ((0, 0), (0, 126)))
    for s_h, d_h in ((srcs[:eh], dsts[:eh]), (srcs[eh:], dsts[eh:])):
        epad = (-eh) % (_CH * _NW)
        s2 = jnp.pad(s_h, (0, epad)).reshape(-1, _CH)
        d2 = jnp.pad(d_h, (0, epad)).reshape(-1, _CH)
        sp, dp = _sc_gather2(mp128, mp128, s2, d2, 128)
        he_h = _edge_enc(eh, sp, dp, w1e_f, r2(b1e_f), w2e, r2(b2e),
                         w3e, r2(b3e), r2(lns_e), r2(lnb_e))
        halves.append({'s2': s2, 'd2': d2, 'd': d_h, 'he': he_h})

    for i in range(len(p['mp_edge'])):
        ew1 = p['mp_edge'][i][0][0]
        ew1sd = jnp.concatenate([ew1[128:256], ew1[256:]], axis=1)
        gsf, gdf = _proj2(h_n, ew1sd)
        aggs = []
        for h in halves:
            gs, gd = _sc_gather2(gsf, gdf, h['s2'], h['d2'], 128)
            h['he'] = _edge_step(
                h['he'], gs, gd, ew1[:128], r2(p['mp_edge'][i][0][1]),
                p['mp_edge'][i][1][0], r2(p['mp_edge'][i][1][1]),
                p['mp_edge'][i][2][0], r2(p['mp_edge'][i][2][1]),
                r2(p['mp_edge_ln'][i][0]), r2(p['mp_edge_ln'][i][1]))
            aggs.append(jax.ops.segment_sum(h['he'], h['d'], num_segments=n))
        agg = aggs[0] + aggs[1]
        h_n = _node_step(h_n, agg, p['mp_node'][i][0][0],
                         r2(p['mp_node'][i][0][1]),
                         p['mp_node'][i][1][0], r2(p['mp_node'][i][1][1]),
                         p['mp_node'][i][2][0], r2(p['mp_node'][i][2][1]),
                         r2(p['mp_node_ln'][i][0]), r2(p['mp_node_ln'][i][1]))

    (w1d, b1d), (w2d, b2d), (w3d, b3d) = p['decoder']
    w3d_f = w3d * p['out_norm_std'][None, :]
    b3d_f = b3d * p['out_norm_std'] + p['out_norm_mean']
    return _decoder3(h_n, w1d, r2(b1d), w2d, r2(b2d), w3d_f, r2(b3d_f))
